# baseline (speedup 1.0000x reference)
import functools

import jax
import jax.numpy as jnp
from jax import lax
from jax.experimental import pallas as pl
from jax.experimental.pallas import tpu as pltpu
from jax.experimental.pallas import tpu_sc as plsc

SENT = 4096
W = 200
D = 64
VOC = 1000
NC, NS = 2, 16
NW = NC * NS
SB = SENT // NW

_mesh = plsc.VectorSubcoreMesh(core_axis_name="c", subcore_axis_name="s")


@functools.partial(
    pl.kernel,
    mesh=_mesh,
    out_type=jax.ShapeDtypeStruct((W, D // 8, NW, 8, SB), jnp.float32),
    scratch_types=[
        pltpu.VMEM((VOC * D,), jnp.float32),
        pltpu.VMEM((SB, W), jnp.int32),
        pltpu.VMEM((2, D // 8, 8, SB), jnp.float32),
        pltpu.SemaphoreType.DMA,
        pltpu.SemaphoreType.DMA,
        pltpu.SemaphoreType.DMA,
    ],
    compiler_params=pltpu.CompilerParams(use_tc_tiling_on_sc=False,
                                         needs_layout_passes=False),
)
def _emb_lookup(idx_hbm, table_hbm, out_hbm, table_v, idx_v, blk_v,
                sem_i, sem_o0, sem_o1):
    wid = lax.axis_index("s") * NC + lax.axis_index("c")
    sems_o = (sem_o0, sem_o1)

    t_copy = pltpu.async_copy(table_hbm, table_v, sem_i)
    pltpu.async_copy(idx_hbm.at[pl.ds(wid * SB, SB)], idx_v, sem_i)
    t_copy.wait()
    pltpu.make_async_copy(idx_hbm.at[pl.ds(0, SB)], idx_v, sem_i).wait()

    lane = lax.iota(jnp.int32, 16)

    def build(w, b):
        wvec = jnp.zeros((16,), jnp.int32) + w
        sidxs = [plsc.load_gather(idx_v, [lane + g * 16, wvec])
                 for g in range(SB // 16)]
        for g in range(SB // 16):
            @plsc.parallel_loop(0, D, unroll=32)
            def _d(d, sidx=sidxs[g], g=g):
                val = plsc.load_gather(table_v, [sidx + d * VOC])
                blk_v[b, d // 8, d % 8, pl.ds(g * 16, 16)] = val

    def flush(w, b):
        pltpu.async_copy(blk_v.at[b], out_hbm.at[w, :, wid], sems_o[b])

    def drain(b):
        pltpu.make_async_copy(blk_v.at[b], out_hbm.at[0, :, wid],
                              sems_o[b]).wait()

    build(0, 0)
    flush(0, 0)

    @pl.loop(1, W - 1, step=2)
    def _pipeline(w0):
        build(w0, 1)
        drain(0)
        flush(w0, 1)
        build(w0 + 1, 0)
        drain(1)
        flush(w0 + 1, 0)

    build(W - 1, 1)
    drain(0)
    flush(W - 1, 1)
    drain(1)


def kernel(inputs, table):
    idx = inputs.astype(jnp.int32)
    out5 = _emb_lookup(idx, table.T.reshape(-1))
    return out5.transpose(2, 4, 0, 1, 3).reshape(SENT, W, D)


# speedup vs baseline: 1.0674x; 1.0674x over previous
import functools

import jax
import jax.numpy as jnp
from jax import lax
from jax.experimental import pallas as pl
from jax.experimental.pallas import tpu as pltpu
from jax.experimental.pallas import tpu_sc as plsc

SENT = 4096
W = 200
D = 64
VOC = 1000
NC, NS = 2, 16
NW = NC * NS
SB = SENT // NW

_mesh = plsc.VectorSubcoreMesh(core_axis_name="c", subcore_axis_name="s")


@functools.partial(
    pl.kernel,
    mesh=_mesh,
    out_type=jax.ShapeDtypeStruct((W, D // 8, NW, 8, SB), jnp.float32),
    scratch_types=[
        pltpu.VMEM((VOC * D,), jnp.float32),
        pltpu.VMEM((SB, W), jnp.int32),
        pltpu.VMEM((2, D // 8, 8, SB), jnp.float32),
        pltpu.SemaphoreType.DMA,
        pltpu.SemaphoreType.DMA,
        pltpu.SemaphoreType.DMA,
    ],
    compiler_params=pltpu.CompilerParams(use_tc_tiling_on_sc=False,
                                         needs_layout_passes=False),
)
def _emb_lookup(idx_hbm, table_hbm, out_hbm, table_v, idx_v, blk_v,
                sem_i, sem_o0, sem_o1):
    wid = lax.axis_index("s") * NC + lax.axis_index("c")
    sems_o = (sem_o0, sem_o1)

    t_copy = pltpu.async_copy(table_hbm, table_v, sem_i)
    pltpu.async_copy(idx_hbm.at[pl.ds(wid * SB, SB)], idx_v, sem_i)
    t_copy.wait()
    pltpu.make_async_copy(idx_hbm.at[pl.ds(0, SB)], idx_v, sem_i).wait()

    lane = lax.iota(jnp.int32, 16)

    def build(w, b):
        wvec = jnp.zeros((16,), jnp.int32) + w
        sidxs = [plsc.load_gather(idx_v, [lane + g * 16, wvec])
                 for g in range(SB // 16)]
        for g in range(SB // 16):
            @plsc.parallel_loop(0, D, unroll=8)
            def _d(d, sidx=sidxs[g], g=g):
                val = plsc.load_gather(table_v, [sidx + d * VOC])
                blk_v[b, d // 8, d % 8, pl.ds(g * 16, 16)] = val

    def flush(w, b):
        pltpu.async_copy(blk_v.at[b], out_hbm.at[w, :, wid], sems_o[b])

    def drain(b):
        pltpu.make_async_copy(blk_v.at[b], out_hbm.at[0, :, wid],
                              sems_o[b]).wait()

    build(0, 0)
    flush(0, 0)

    @pl.loop(1, W - 1, step=2)
    def _pipeline(w0):
        build(w0, 1)
        drain(0)
        flush(w0, 1)
        build(w0 + 1, 0)
        drain(1)
        flush(w0 + 1, 0)

    build(W - 1, 1)
    drain(0)
    flush(W - 1, 1)
    drain(1)


def kernel(inputs, table):
    idx = inputs.astype(jnp.int32)
    out5 = _emb_lookup(idx, table.T.reshape(-1))
    return out5.transpose(2, 4, 0, 1, 3).reshape(SENT, W, D)
